# Initial kernel scaffold; baseline (speedup 1.0000x reference)
import jax
import jax.numpy as jnp
from jax.experimental import pallas as pl


def kernel(x, ptr, ptr_table):
    raise NotImplementedError("write your pallas kernel here")


# speedup vs baseline: 1.2527x; 1.2527x over previous
import functools

import jax
import jax.numpy as jnp
from jax import lax
from jax.experimental import pallas as pl
from jax.experimental.pallas import tpu as pltpu
from jax.experimental.pallas import tpu_sc as plsc

B, S, D = 4, 8192, 1024
N = B * S
L = 16
NC, NS = 2, 16
NW = NC * NS
TOK_PER_W = N // NW
T = 64
NCHUNK = TOK_PER_W // T
CG = 8
CGW = D // CG


def _sc_kernel():
    mesh = plsc.VectorSubcoreMesh(core_axis_name="c", subcore_axis_name="s")

    @functools.partial(
        pl.kernel,
        mesh=mesh,
        out_type=jax.ShapeDtypeStruct((N, D), jnp.float32),
        scratch_types=[
            pltpu.VMEM((TOK_PER_W,), jnp.int32),
            pltpu.VMEM((TOK_PER_W,), jnp.float32),
            pltpu.VMEM((2, D), jnp.float32),
            pltpu.VMEM((D,), jnp.float32),
            pltpu.VMEM((T, D), jnp.float32),
        ],
    )
    def k(x_hbm, ptr_hbm, tab_hbm, out_hbm, ptri_v, ptrf_v, tab_v, diff_v, buf):
        wid = lax.axis_index("s") * NC + lax.axis_index("c")
        tok0 = pl.multiple_of(wid * TOK_PER_W, TOK_PER_W)

        pltpu.sync_copy(ptr_hbm.at[pl.ds(tok0, TOK_PER_W)], ptri_v)
        pltpu.sync_copy(tab_hbm, tab_v)
        for c in range(TOK_PER_W // L):
            sl = pl.ds(c * L, L)
            ptrf_v[sl] = ptri_v[sl].astype(jnp.float32)
        for c in range(D // L):
            sl = pl.ds(c * L, L)
            diff_v[sl] = tab_v[1, sl] - tab_v[0, sl]

        def chunk_body(i, _):
            row0 = tok0 + i * T
            pltpu.sync_copy(x_hbm.at[pl.ds(row0, T)], buf)

            for cg in range(CG):
                r0s = [tab_v[0, pl.ds(cg * CGW + j * L, L)] for j in range(8)]
                dfs = [diff_v[pl.ds(cg * CGW + j * L, L)] for j in range(8)]

                def grp_body(g, _):
                    pv = ptrf_v[pl.ds(i * T + g * L, L)]
                    for tt in range(L):
                        pf = pv[tt]
                        t = g * L + tt
                        for j in range(8):
                            sl = pl.ds(cg * CGW + j * L, L)
                            buf[t, sl] = buf[t, sl] + (r0s[j] + pf * dfs[j])
                    return 0

                lax.fori_loop(0, T // L, grp_body, 0)

            pltpu.sync_copy(buf, out_hbm.at[pl.ds(row0, T)])
            return 0

        lax.fori_loop(0, NCHUNK, chunk_body, 0)

    return k


_sc_call = _sc_kernel()


@jax.jit
def kernel(x, ptr, ptr_table):
    x2 = x.reshape(N, D)
    p1 = ptr.astype(jnp.int32).reshape(N)
    out = _sc_call(x2, p1, ptr_table)
    return out.reshape(B, S, D)


# speedup vs baseline: 2.2538x; 1.7992x over previous
import functools

import jax
import jax.numpy as jnp
from jax import lax
from jax.experimental import pallas as pl
from jax.experimental.pallas import tpu as pltpu
from jax.experimental.pallas import tpu_sc as plsc

B, S, D = 4, 8192, 1024
N = B * S
L = 16
NC, NS = 2, 16
NW = NC * NS
TOK_PER_W = N // NW
T = 16
NCHUNK = TOK_PER_W // T
CG = 8
CGW = D // CG


def _sc_kernel():
    mesh = plsc.VectorSubcoreMesh(core_axis_name="c", subcore_axis_name="s")

    @functools.partial(
        pl.kernel,
        mesh=mesh,
        out_type=jax.ShapeDtypeStruct((N, D), jnp.float32),
        scratch_types=[
            pltpu.VMEM((TOK_PER_W,), jnp.float32),
            pltpu.VMEM((TOK_PER_W,), jnp.int32),
            pltpu.VMEM((2, D), jnp.float32),
            pltpu.VMEM((D,), jnp.float32),
            pltpu.VMEM((T, D), jnp.float32),
            pltpu.VMEM((T, D), jnp.float32),
            pltpu.VMEM((T, D), jnp.float32),
            pltpu.VMEM((T, D), jnp.float32),
            pltpu.SemaphoreType.DMA,
            pltpu.SemaphoreType.DMA,
            pltpu.SemaphoreType.DMA,
            pltpu.SemaphoreType.DMA,
        ],
    )
    def k(x_hbm, ptr_hbm, tab_hbm, out_hbm, ptrf_v, ptri_v, tab_v, diff_v,
          in0, in1, ot0, ot1, isem0, isem1, osem0, osem1):
        inb = (in0, in1)
        otb = (ot0, ot1)
        isem = (isem0, isem1)
        osem = (osem0, osem1)

        wid = lax.axis_index("s") * NC + lax.axis_index("c")
        tok0 = pl.multiple_of(wid * TOK_PER_W, TOK_PER_W)

        pltpu.sync_copy(ptr_hbm.at[pl.ds(tok0, TOK_PER_W)], ptri_v)
        pltpu.sync_copy(tab_hbm, tab_v)
        for c in range(TOK_PER_W // L):
            sl = pl.ds(c * L, L)
            ptrf_v[sl] = ptri_v[sl].astype(jnp.float32)
        for c in range(D // L):
            sl = pl.ds(c * L, L)
            diff_v[sl] = tab_v[1, sl] - tab_v[0, sl]

        for b in range(2):
            pltpu.async_copy(x_hbm.at[pl.ds(tok0 + b * T, T)], inb[b], isem[b])

        def grp_body(i2, _):
            for b in range(2):
                i = i2 * 2 + b
                row = tok0 + i * T
                pltpu.make_async_copy(
                    x_hbm.at[pl.ds(row, T)], inb[b], isem[b]).wait()
                @pl.when(i2 > 0)
                def _():
                    pltpu.make_async_copy(
                        otb[b], out_hbm.at[pl.ds(row, T)], osem[b]).wait()

                pv = ptrf_v[pl.ds(i * T, L)]
                pfs = [pv[tt] for tt in range(L)]

                def cg_body(cg, _):
                    col = pl.multiple_of(cg * CGW, CGW)
                    r0s = [tab_v[0, pl.ds(col + j * L, L)] for j in range(8)]
                    dfs = [diff_v[pl.ds(col + j * L, L)] for j in range(8)]
                    for tt in range(T):
                        for j in range(8):
                            sl = pl.ds(col + j * L, L)
                            otb[b][tt, sl] = inb[b][tt, sl] + (
                                r0s[j] + pfs[tt] * dfs[j])
                    return 0

                lax.fori_loop(0, CG, cg_body, 0)

                pltpu.async_copy(otb[b], out_hbm.at[pl.ds(row, T)], osem[b])

                @pl.when(i + 2 < NCHUNK)
                def _():
                    pltpu.async_copy(
                        x_hbm.at[pl.ds(row + 2 * T, T)], inb[b], isem[b])
            return 0

        lax.fori_loop(0, NCHUNK // 2, grp_body, 0)

        for b in range(2):
            row = tok0 + (NCHUNK - 2 + b) * T
            pltpu.make_async_copy(
                otb[b], out_hbm.at[pl.ds(row, T)], osem[b]).wait()

    return k


_sc_call = _sc_kernel()


@jax.jit
def kernel(x, ptr, ptr_table):
    x2 = x.reshape(N, D)
    p1 = ptr.astype(jnp.int32).reshape(N)
    out = _sc_call(x2, p1, ptr_table)
    return out.reshape(B, S, D)
